# baseline (speedup 1.0000x reference)
import functools

import jax
import jax.numpy as jnp
from jax import lax
from jax.experimental import pallas as pl
from jax.experimental.pallas import tpu as pltpu
from jax.experimental.pallas import tpu_sc as plsc

VOC = (1958, 1430, 131)
KG_LEAF = VOC[0] + VOC[1] + VOC[2]
N_NODES = 10000
N_ENT = N_NODES - KG_LEAF
N_REL = 17
EMB = 128
HOPS = 3
B, S = 16, 8
DN, PN, MN = 40, 32, 45
MED_PAD = VOC[2] + 2
NTOK = B * S * (DN + PN + MN)
TB = 128

NE = 320000
NNZ = 28152
NC, NS = 2, 16
NW = NC * NS
CH = 128

NCH0_E, NCH1_E = 136, 24
NCHT_E = NS * (NCH0_E + NCH1_E)
NCHT_E_PAD = NCHT_E + NCH0_E
NCH0_U, NCH1_U = 14, 2
NCHT_U = NS * (NCH0_U + NCH1_U)
NCHT_U_PAD = NCHT_U + NCH0_U

ENT_PAD = 6528
LEAF_PAD = 3584
ROWS_T = ENT_PAD // NS
ROWS_TU = LEAF_PAD // NS

_MESH = plsc.VectorSubcoreMesh(core_axis_name="c", subcore_axis_name="s",
                               num_cores=NC, num_subcores=NS)
_SC_PARAMS = pltpu.CompilerParams(use_tc_tiling_on_sc=False)


def _copy_rows(src, dst, base, nrows):
    for k in range(nrows // CH):
        sl = pl.ds(base + k * CH, CH)
        pltpu.sync_copy(src.at[sl], dst.at[sl])
    rem = nrows % CH
    if rem:
        sl = pl.ds(base + (nrows // CH) * CH, rem)
        pltpu.sync_copy(src.at[sl], dst.at[sl])


def _edge_body(with_deg, p_hbm, gidx_hbm, head_hbm, zeros_hbm,
               part_hbm, degp_hbm, gidx_v, head_v,
               buf_a, buf_b, ones_v, zmini_v, sem_a, sem_b, acc, degacc):
    c = lax.axis_index("c")
    s = lax.axis_index("s")
    base_ch = jnp.where(c == 0, s * NCH0_E, NS * NCH0_E + s * NCH1_E)
    cnt = jnp.where(c == 0, NCH0_E, NCH1_E)

    pltpu.sync_copy(zeros_hbm.at[pl.ds(s * ROWS_T, ROWS_T)],
                    acc.at[pl.ds(s * ROWS_T, ROWS_T)])
    pltpu.sync_copy(gidx_hbm.at[pl.ds(base_ch, NCH0_E)], gidx_v)
    pltpu.sync_copy(head_hbm.at[pl.ds(base_ch, NCH0_E)], head_v)

    if with_deg:
        one16 = jnp.ones((16,), jnp.float32)
        z16 = jnp.zeros((16,), jnp.float32)

        def _ones(i, _):
            ones_v[i, :] = one16
            zmini_v[i, :] = z16
            return 0
        lax.fori_loop(0, CH, _ones, 0)
        for k in range(ROWS_T // CH):
            pltpu.sync_copy(zmini_v,
                            degacc.at[pl.ds(s * ROWS_T + k * CH, CH)])
        if ROWS_T % CH:
            pltpu.sync_copy(zmini_v.at[pl.ds(0, ROWS_T % CH)],
                            degacc.at[pl.ds(s * ROWS_T
                                            + (ROWS_T // CH) * CH,
                                            ROWS_T % CH)])

    plsc.subcore_barrier()

    bufs = ((buf_a, sem_a), (buf_b, sem_b))
    pltpu.async_copy(p_hbm.at[gidx_v.at[0]], buf_a, sem_a)
    pltpu.async_copy(p_hbm.at[gidx_v.at[1]], buf_b, sem_b)

    def _chunks(j2, _):
        for b, (buf, sem) in enumerate(bufs):
            ch = j2 * 2 + b
            pltpu.make_async_copy(p_hbm.at[pl.ds(0, CH)], buf, sem).wait()
            pltpu.sync_copy(buf, acc.at[head_v.at[ch]], add=True)
            if with_deg:
                pltpu.sync_copy(ones_v, degacc.at[head_v.at[ch]], add=True)

            @pl.when(ch + 2 < cnt)
            def _():
                pltpu.async_copy(p_hbm.at[gidx_v.at[ch + 2]], buf, sem)
        return 0
    lax.fori_loop(0, cnt // 2, _chunks, 0)

    plsc.subcore_barrier()

    _copy_rows(acc, part_hbm.at[c], s * ROWS_T, ROWS_T)
    if with_deg:
        _copy_rows(degacc, degp_hbm.at[c], s * ROWS_T, ROWS_T)


def _make_edge_kernel(with_deg):
    if with_deg:
        out_type = [jax.ShapeDtypeStruct((NC, ENT_PAD, EMB), jnp.float32),
                    jax.ShapeDtypeStruct((NC, ENT_PAD, 16), jnp.float32)]
        body = functools.partial(_edge_body, True)
    else:
        out_type = jax.ShapeDtypeStruct((NC, ENT_PAD, EMB), jnp.float32)

        def body(p, g, h, z, part, *rest):
            return _edge_body(False, p, g, h, z, part, None, *rest)
    scratch = [
        pltpu.VMEM((NCH0_E, CH), jnp.int32),
        pltpu.VMEM((NCH0_E, CH), jnp.int32),
        pltpu.VMEM((CH, EMB), jnp.float32),
        pltpu.VMEM((CH, EMB), jnp.float32),
        pltpu.VMEM((CH, 16), jnp.float32),
        pltpu.VMEM((CH, 16), jnp.float32),
        pltpu.SemaphoreType.DMA,
        pltpu.SemaphoreType.DMA,
        pltpu.VMEM_SHARED((ENT_PAD, EMB), jnp.float32),
        pltpu.VMEM_SHARED((ENT_PAD, 16), jnp.float32),
    ]
    return pl.kernel(body, out_type=out_type, mesh=_MESH,
                     scratch_types=scratch, compiler_params=_SC_PARAMS)


def _user_body(ent_hbm, col_hbm, row_hbm, valb_hbm, zeros_hbm, part_hbm,
               col_v, row_v, valb_v, buf_a, buf_b, sem_a, sem_b, acc):
    c = lax.axis_index("c")
    s = lax.axis_index("s")
    base_ch = jnp.where(c == 0, s * NCH0_U, NS * NCH0_U + s * NCH1_U)
    cnt = jnp.where(c == 0, NCH0_U, NCH1_U)

    pltpu.sync_copy(zeros_hbm.at[pl.ds(s * ROWS_TU, ROWS_TU)],
                    acc.at[pl.ds(s * ROWS_TU, ROWS_TU)])
    pltpu.sync_copy(col_hbm.at[pl.ds(base_ch, NCH0_U)], col_v)
    pltpu.sync_copy(row_hbm.at[pl.ds(base_ch, NCH0_U)], row_v)
    pltpu.sync_copy(valb_hbm.at[pl.ds(base_ch * CH, NCH0_U * CH)], valb_v)
    plsc.subcore_barrier()

    bufs = ((buf_a, sem_a), (buf_b, sem_b))
    pltpu.async_copy(ent_hbm.at[col_v.at[0]], buf_a, sem_a)
    pltpu.async_copy(ent_hbm.at[col_v.at[1]], buf_b, sem_b)

    def _chunks(j2, _):
        for b, (buf, sem) in enumerate(bufs):
            ch = j2 * 2 + b
            pltpu.make_async_copy(ent_hbm.at[pl.ds(0, CH)], buf, sem).wait()

            def _scale(e, _):
                v16 = valb_v[ch * CH + e, :]
                for d in range(EMB // 16):
                    sl = pl.ds(d * 16, 16)
                    buf[e, sl] = buf[e, sl] * v16
                return 0
            lax.fori_loop(0, CH, _scale, 0)
            pltpu.sync_copy(buf, acc.at[row_v.at[ch]], add=True)

            @pl.when(ch + 2 < cnt)
            def _():
                pltpu.async_copy(ent_hbm.at[col_v.at[ch + 2]], buf, sem)
        return 0
    lax.fori_loop(0, cnt // 2, _chunks, 0)

    plsc.subcore_barrier()
    _copy_rows(acc, part_hbm.at[c], s * ROWS_TU, ROWS_TU)


_user_kernel = pl.kernel(
    _user_body,
    out_type=jax.ShapeDtypeStruct((NC, LEAF_PAD, EMB), jnp.float32),
    mesh=_MESH,
    scratch_types=[
        pltpu.VMEM((NCH0_U, CH), jnp.int32),
        pltpu.VMEM((NCH0_U, CH), jnp.int32),
        pltpu.VMEM((NCH0_U * CH, 16), jnp.float32),
        pltpu.VMEM((CH, EMB), jnp.float32),
        pltpu.VMEM((CH, EMB), jnp.float32),
        pltpu.SemaphoreType.DMA,
        pltpu.SemaphoreType.DMA,
        pltpu.VMEM_SHARED((LEAF_PAD, EMB), jnp.float32),
    ],
    compiler_params=_SC_PARAMS)


def _gidx_body(tail_ref, rel_ref, g_ref):
    g_ref[...] = (rel_ref[...] - 1) * ENT_PAD + tail_ref[...]


def _gidx_kernel(tail2, rel2):
    return pl.pallas_call(
        _gidx_body,
        out_shape=jax.ShapeDtypeStruct((NCHT_E_PAD, CH), jnp.int32),
    )(tail2, rel2)


def _prescale_body(cur_ref, relw_ref, p_ref):
    p_ref[0] = cur_ref[...] * relw_ref[0]


def _prescale(cur, rel_weight):
    return pl.pallas_call(
        _prescale_body,
        grid=(N_REL - 1,),
        in_specs=[
            pl.BlockSpec((ENT_PAD, EMB), lambda r: (0, 0)),
            pl.BlockSpec((1, 1, EMB), lambda r: (r, 0, 0)),
        ],
        out_specs=pl.BlockSpec((1, ENT_PAD, EMB), lambda r: (r, 0, 0)),
        out_shape=jax.ShapeDtypeStruct((N_REL - 1, ENT_PAD, EMB),
                                       jnp.float32),
    )(cur, rel_weight.reshape(N_REL - 1, 1, EMB))


def _merge_ent1_body(part_ref, degp_ref, cur_ref, deg_ref):
    p = part_ref[0] + part_ref[1]
    d = degp_ref[...]
    deg = jnp.maximum(d[0, :, 0] + d[1, :, 0], 1.0)
    agg = p / deg[:, None]
    n = jnp.sqrt(jnp.sum(agg * agg, axis=1, keepdims=True))
    cur_ref[...] = agg / (n + 1e-8)
    deg_ref[...] = jnp.broadcast_to(deg[:, None], (ENT_PAD, 8))


def _merge_ent1(part, degp):
    return pl.pallas_call(
        _merge_ent1_body,
        out_shape=[jax.ShapeDtypeStruct((ENT_PAD, EMB), jnp.float32),
                   jax.ShapeDtypeStruct((ENT_PAD, 8), jnp.float32)],
    )(part, degp)


def _merge_ent2_body(part_ref, deg_ref, cur_ref):
    p = part_ref[0] + part_ref[1]
    agg = p / deg_ref[:, :1]
    n = jnp.sqrt(jnp.sum(agg * agg, axis=1, keepdims=True))
    cur_ref[...] = agg / (n + 1e-8)


def _merge_ent2(part, deg):
    return pl.pallas_call(
        _merge_ent2_body,
        out_shape=jax.ShapeDtypeStruct((ENT_PAD, EMB), jnp.float32),
    )(part, deg)


def _merge_user_body(part_ref, cu_ref, ures_ref, latent_ref, att_ref,
                     relw_ref, cu_out_ref, ures_out_ref):
    ua = part_ref[0] + part_ref[1]
    cu_prev = cu_ref[...]
    logits = lax.dot_general(cu_prev, latent_ref[...],
                             (((1,), (1,)), ((), ())),
                             preferred_element_type=jnp.float32)
    m = jnp.max(logits, axis=1, keepdims=True)
    e = jnp.exp(logits - m)
    score = e / jnp.sum(e, axis=1, keepdims=True)
    a = att_ref[...]
    am = jnp.max(a, axis=1, keepdims=True)
    ae = jnp.exp(a - am)
    aw = ae / jnp.sum(ae, axis=1, keepdims=True)
    disen_w = jnp.dot(aw, relw_ref[...], preferred_element_type=jnp.float32)
    scale = jnp.dot(score, disen_w, preferred_element_type=jnp.float32)
    ua = ua * scale + ua
    n = jnp.sqrt(jnp.sum(ua * ua, axis=1, keepdims=True))
    cu = ua / (n + 1e-8)
    cu_out_ref[...] = cu
    ures_out_ref[...] = ures_ref[...] + cu


def _merge_user(part, cu_prev, ures_prev, latent_emb, att, rel_weight):
    return pl.pallas_call(
        _merge_user_body,
        out_shape=[jax.ShapeDtypeStruct((LEAF_PAD, EMB), jnp.float32),
                   jax.ShapeDtypeStruct((LEAF_PAD, EMB), jnp.float32)],
    )(part, cu_prev, ures_prev, latent_emb, att, rel_weight)


def _attn_body(x_ref, lt_ref, l_ref, kmask_ref, tmult_ref, o_ref):
    x = x_ref[...]
    sc = jnp.dot(x, lt_ref[...], preferred_element_type=jnp.float32)
    sc = sc + kmask_ref[...]
    m = jnp.max(sc, axis=1, keepdims=True)
    p = jnp.exp(sc - m)
    den = jnp.sum(p, axis=1, keepdims=True)
    w = p / den
    o = jnp.dot(w, l_ref[...], preferred_element_type=jnp.float32)
    o_ref[...] = o * tmult_ref[...]


def _intent_attention(x, lpad, tok_mult):
    kmask = jnp.where(jnp.arange(LEAF_PAD) < KG_LEAF, 0.0, -1e30)[None, :]
    grid = NTOK // TB
    return pl.pallas_call(
        _attn_body,
        grid=(grid,),
        in_specs=[
            pl.BlockSpec((TB, EMB), lambda i: (i, 0)),
            pl.BlockSpec((EMB, LEAF_PAD), lambda i: (0, 0)),
            pl.BlockSpec((LEAF_PAD, EMB), lambda i: (0, 0)),
            pl.BlockSpec((1, LEAF_PAD), lambda i: (0, 0)),
            pl.BlockSpec((TB, 1), lambda i: (i, 0)),
        ],
        out_specs=pl.BlockSpec((TB, EMB), lambda i: (i, 0)),
        out_shape=jax.ShapeDtypeStruct((NTOK, EMB), jnp.float32),
    )(x, lpad.T, lpad, kmask, tok_mult)


def _pad_chunks(x, nch_pad, fill):
    xx = x if x.dtype == jnp.float32 else x.astype(jnp.int32)
    pad = jnp.full((nch_pad * CH - xx.shape[0],), fill, xx.dtype)
    return jnp.concatenate([xx, pad]).reshape(nch_pad, CH)


def kernel(diag_table, proc_table, med_table, all_embed, latent_emb,
           rel_weight, disen_weight_att, inter_val, d_mask, p_mask, m_mask,
           diseases, procedures, medications, edge_index, edge_type,
           inter_row, inter_col):
    head = edge_index[0]
    tail = edge_index[1]
    tail2 = _pad_chunks(tail, NCHT_E_PAD, N_ENT)
    rel2 = _pad_chunks(edge_type, NCHT_E_PAD, 1)
    head2 = _pad_chunks(head, NCHT_E_PAD, N_ENT)
    col2 = _pad_chunks(inter_col, NCHT_U_PAD, 0)
    row2 = _pad_chunks(inter_row, NCHT_U_PAD, KG_LEAF)
    val_p = jnp.concatenate(
        [inter_val, jnp.zeros((NCHT_U_PAD * CH - NNZ,), jnp.float32)])
    valb = jnp.broadcast_to(val_p[:, None], (NCHT_U_PAD * CH, 16))

    gidx2 = _gidx_kernel(tail2, rel2)

    ent0 = jnp.zeros((ENT_PAD, EMB), jnp.float32).at[:N_ENT].set(
        all_embed[KG_LEAF:])
    user0 = jnp.zeros((LEAF_PAD, EMB), jnp.float32).at[:KG_LEAF].set(
        all_embed[:KG_LEAF])
    zeros = jnp.zeros((ENT_PAD, EMB), jnp.float32)

    edge1 = _make_edge_kernel(True)
    edge2 = _make_edge_kernel(False)

    p0 = _prescale(ent0, rel_weight).reshape((N_REL - 1) * ENT_PAD, EMB)
    e_part1, degp = edge1(p0, gidx2, head2, zeros)
    u_part1 = _user_kernel(ent0, col2, row2, valb, zeros)
    cur_ent1, deg8 = _merge_ent1(e_part1, degp)
    cur_user1, user_res = _merge_user(u_part1, user0, user0, latent_emb,
                                      disen_weight_att, rel_weight)
    p1 = _prescale(cur_ent1, rel_weight).reshape((N_REL - 1) * ENT_PAD, EMB)
    e_part2 = edge2(p1, gidx2, head2, zeros)
    u_part2 = _user_kernel(cur_ent1, col2, row2, valb, zeros)
    cur_ent2 = _merge_ent2(e_part2, deg8)
    cur_user2, user_res = _merge_user(u_part2, cur_user1, user_res,
                                      latent_emb, disen_weight_att,
                                      rel_weight)
    u_part3 = _user_kernel(cur_ent2, col2, row2, valb, zeros)
    _, user_res = _merge_user(u_part3, cur_user2, user_res, latent_emb,
                              disen_weight_att, rel_weight)

    new_med = jnp.concatenate(
        [jnp.full((B, 1, MN), MED_PAD, dtype=medications.dtype),
         medications[:, :-1, :]], axis=1)
    new_m_mask = jnp.concatenate(
        [jnp.full((B, 1, MN), -1e9, dtype=jnp.float32),
         m_mask[:, :-1, :]], axis=1)

    diag_emb = diag_table[diseases].reshape(B * S, DN, EMB)
    proc_emb = proc_table[procedures].reshape(B * S, PN, EMB)
    med_emb = med_table[new_med].reshape(B * S, MN, EMB)
    x = jnp.concatenate([diag_emb, proc_emb, med_emb], axis=1)
    x = x.reshape(NTOK, EMB)

    masks = jnp.concatenate([d_mask.reshape(B * S, DN),
                             p_mask.reshape(B * S, PN),
                             new_m_mask.reshape(B * S, MN)], axis=1)
    tok_mult = jnp.where(masks.reshape(NTOK, 1) != 0, 0.0, 1.0)

    out = _intent_attention(x, user_res, tok_mult)
    return out.reshape(B * S, DN + PN + MN, EMB)


# speedup vs baseline: 1.0405x; 1.0405x over previous
import functools

import jax
import jax.numpy as jnp
from jax import lax
from jax.experimental import pallas as pl
from jax.experimental.pallas import tpu as pltpu
from jax.experimental.pallas import tpu_sc as plsc

VOC = (1958, 1430, 131)
KG_LEAF = VOC[0] + VOC[1] + VOC[2]
N_NODES = 10000
N_ENT = N_NODES - KG_LEAF
N_REL = 17
EMB = 128
HOPS = 3
B, S = 16, 8
DN, PN, MN = 40, 32, 45
MED_PAD = VOC[2] + 2
NTOK = B * S * (DN + PN + MN)
TB = 128

NE = 320000
NNZ = 28152
NC, NS = 2, 16
NW = NC * NS
CH = 128

NCH0_E, NCH1_E = 120, 40
NCHT_E = NS * (NCH0_E + NCH1_E)
NCHT_E_PAD = NCHT_E + NCH0_E
NCH0_U, NCH1_U = 14, 2
NCHT_U = NS * (NCH0_U + NCH1_U)
NCHT_U_PAD = NCHT_U + NCH0_U

ENT_PAD = 6528
LEAF_PAD = 3584
ROWS_T = ENT_PAD // NS
ROWS_TU = LEAF_PAD // NS

_MESH = plsc.VectorSubcoreMesh(core_axis_name="c", subcore_axis_name="s",
                               num_cores=NC, num_subcores=NS)
_SC_PARAMS = pltpu.CompilerParams(use_tc_tiling_on_sc=False)


def _copy_rows(src, dst, base, nrows):
    for k in range(nrows // CH):
        sl = pl.ds(base + k * CH, CH)
        pltpu.sync_copy(src.at[sl], dst.at[sl])
    rem = nrows % CH
    if rem:
        sl = pl.ds(base + (nrows // CH) * CH, rem)
        pltpu.sync_copy(src.at[sl], dst.at[sl])


def _edge_body(with_deg, p_hbm, gidx_hbm, head_hbm, zeros_hbm,
               part_hbm, degp_hbm, gidx_v, head_v,
               buf_a, buf_b, ones_v, zmini_v, sem_a, sem_b, acc, degacc):
    c = lax.axis_index("c")
    s = lax.axis_index("s")
    base_ch = jnp.where(c == 0, s * NCH0_E, NS * NCH0_E + s * NCH1_E)
    cnt = jnp.where(c == 0, NCH0_E, NCH1_E)

    pltpu.sync_copy(zeros_hbm.at[pl.ds(s * ROWS_T, ROWS_T)],
                    acc.at[pl.ds(s * ROWS_T, ROWS_T)])
    pltpu.sync_copy(gidx_hbm.at[pl.ds(base_ch, NCH0_E)], gidx_v)
    pltpu.sync_copy(head_hbm.at[pl.ds(base_ch, NCH0_E)], head_v)

    if with_deg:
        one16 = jnp.ones((16,), jnp.float32)
        z16 = jnp.zeros((16,), jnp.float32)

        def _ones(i, _):
            ones_v[i, :] = one16
            zmini_v[i, :] = z16
            return 0
        lax.fori_loop(0, CH, _ones, 0)
        for k in range(ROWS_T // CH):
            pltpu.sync_copy(zmini_v,
                            degacc.at[pl.ds(s * ROWS_T + k * CH, CH)])
        if ROWS_T % CH:
            pltpu.sync_copy(zmini_v.at[pl.ds(0, ROWS_T % CH)],
                            degacc.at[pl.ds(s * ROWS_T
                                            + (ROWS_T // CH) * CH,
                                            ROWS_T % CH)])

    plsc.subcore_barrier()

    bufs = ((buf_a, sem_a), (buf_b, sem_b))
    pltpu.async_copy(p_hbm.at[gidx_v.at[0]], buf_a, sem_a)
    pltpu.async_copy(p_hbm.at[gidx_v.at[1]], buf_b, sem_b)

    def _chunks(j2, _):
        for b, (buf, sem) in enumerate(bufs):
            ch = j2 * 2 + b
            pltpu.make_async_copy(p_hbm.at[pl.ds(0, CH)], buf, sem).wait()
            pltpu.sync_copy(buf, acc.at[head_v.at[ch]], add=True)
            if with_deg:
                pltpu.sync_copy(ones_v, degacc.at[head_v.at[ch]], add=True)

            @pl.when(ch + 2 < cnt)
            def _():
                pltpu.async_copy(p_hbm.at[gidx_v.at[ch + 2]], buf, sem)
        return 0
    lax.fori_loop(0, cnt // 2, _chunks, 0)

    plsc.subcore_barrier()

    _copy_rows(acc, part_hbm.at[c], s * ROWS_T, ROWS_T)
    if with_deg:
        _copy_rows(degacc, degp_hbm.at[c], s * ROWS_T, ROWS_T)


def _make_edge_kernel(with_deg):
    if with_deg:
        out_type = [jax.ShapeDtypeStruct((NC, ENT_PAD, EMB), jnp.float32),
                    jax.ShapeDtypeStruct((NC, ENT_PAD, 16), jnp.float32)]
        body = functools.partial(_edge_body, True)
    else:
        out_type = jax.ShapeDtypeStruct((NC, ENT_PAD, EMB), jnp.float32)

        def body(p, g, h, z, part, *rest):
            return _edge_body(False, p, g, h, z, part, None, *rest)
    scratch = [
        pltpu.VMEM((NCH0_E, CH), jnp.int32),
        pltpu.VMEM((NCH0_E, CH), jnp.int32),
        pltpu.VMEM((CH, EMB), jnp.float32),
        pltpu.VMEM((CH, EMB), jnp.float32),
        pltpu.VMEM((CH, 16), jnp.float32),
        pltpu.VMEM((CH, 16), jnp.float32),
        pltpu.SemaphoreType.DMA,
        pltpu.SemaphoreType.DMA,
        pltpu.VMEM_SHARED((ENT_PAD, EMB), jnp.float32),
        pltpu.VMEM_SHARED((ENT_PAD, 16), jnp.float32),
    ]
    return pl.kernel(body, out_type=out_type, mesh=_MESH,
                     scratch_types=scratch, compiler_params=_SC_PARAMS)


def _user_body(ent_hbm, col_hbm, row_hbm, valb_hbm, zeros_hbm, part_hbm,
               col_v, row_v, valb_v, buf_a, buf_b, sem_a, sem_b, acc):
    c = lax.axis_index("c")
    s = lax.axis_index("s")
    base_ch = jnp.where(c == 0, s * NCH0_U, NS * NCH0_U + s * NCH1_U)
    cnt = jnp.where(c == 0, NCH0_U, NCH1_U)

    pltpu.sync_copy(zeros_hbm.at[pl.ds(s * ROWS_TU, ROWS_TU)],
                    acc.at[pl.ds(s * ROWS_TU, ROWS_TU)])
    pltpu.sync_copy(col_hbm.at[pl.ds(base_ch, NCH0_U)], col_v)
    pltpu.sync_copy(row_hbm.at[pl.ds(base_ch, NCH0_U)], row_v)
    pltpu.sync_copy(valb_hbm.at[pl.ds(base_ch * CH, NCH0_U * CH)], valb_v)
    plsc.subcore_barrier()

    bufs = ((buf_a, sem_a), (buf_b, sem_b))
    pltpu.async_copy(ent_hbm.at[col_v.at[0]], buf_a, sem_a)
    pltpu.async_copy(ent_hbm.at[col_v.at[1]], buf_b, sem_b)

    def _chunks(j2, _):
        for b, (buf, sem) in enumerate(bufs):
            ch = j2 * 2 + b
            pltpu.make_async_copy(ent_hbm.at[pl.ds(0, CH)], buf, sem).wait()

            def _scale(e, _):
                v16 = valb_v[ch * CH + e, :]
                for d in range(EMB // 16):
                    sl = pl.ds(d * 16, 16)
                    buf[e, sl] = buf[e, sl] * v16
                return 0
            lax.fori_loop(0, CH, _scale, 0)
            pltpu.sync_copy(buf, acc.at[row_v.at[ch]], add=True)

            @pl.when(ch + 2 < cnt)
            def _():
                pltpu.async_copy(ent_hbm.at[col_v.at[ch + 2]], buf, sem)
        return 0
    lax.fori_loop(0, cnt // 2, _chunks, 0)

    plsc.subcore_barrier()
    _copy_rows(acc, part_hbm.at[c], s * ROWS_TU, ROWS_TU)


_user_kernel = pl.kernel(
    _user_body,
    out_type=jax.ShapeDtypeStruct((NC, LEAF_PAD, EMB), jnp.float32),
    mesh=_MESH,
    scratch_types=[
        pltpu.VMEM((NCH0_U, CH), jnp.int32),
        pltpu.VMEM((NCH0_U, CH), jnp.int32),
        pltpu.VMEM((NCH0_U * CH, 16), jnp.float32),
        pltpu.VMEM((CH, EMB), jnp.float32),
        pltpu.VMEM((CH, EMB), jnp.float32),
        pltpu.SemaphoreType.DMA,
        pltpu.SemaphoreType.DMA,
        pltpu.VMEM_SHARED((LEAF_PAD, EMB), jnp.float32),
    ],
    compiler_params=_SC_PARAMS)


def _gidx_body(tail_ref, rel_ref, g_ref):
    g_ref[...] = (rel_ref[...] - 1) * ENT_PAD + tail_ref[...]


def _gidx_kernel(tail2, rel2):
    return pl.pallas_call(
        _gidx_body,
        out_shape=jax.ShapeDtypeStruct((NCHT_E_PAD, CH), jnp.int32),
    )(tail2, rel2)


def _prescale_body(cur_ref, relw_ref, p_ref):
    p_ref[0] = cur_ref[...] * relw_ref[0]


def _prescale(cur, rel_weight):
    return pl.pallas_call(
        _prescale_body,
        grid=(N_REL - 1,),
        in_specs=[
            pl.BlockSpec((ENT_PAD, EMB), lambda r: (0, 0)),
            pl.BlockSpec((1, 1, EMB), lambda r: (r, 0, 0)),
        ],
        out_specs=pl.BlockSpec((1, ENT_PAD, EMB), lambda r: (r, 0, 0)),
        out_shape=jax.ShapeDtypeStruct((N_REL - 1, ENT_PAD, EMB),
                                       jnp.float32),
    )(cur, rel_weight.reshape(N_REL - 1, 1, EMB))


def _merge_ent1_body(part_ref, degp_ref, cur_ref, deg_ref):
    p = part_ref[0] + part_ref[1]
    d = degp_ref[...]
    deg = jnp.maximum(d[0, :, 0] + d[1, :, 0], 1.0)
    agg = p / deg[:, None]
    n = jnp.sqrt(jnp.sum(agg * agg, axis=1, keepdims=True))
    cur_ref[...] = agg / (n + 1e-8)
    deg_ref[...] = jnp.broadcast_to(deg[:, None], (ENT_PAD, 8))


def _merge_ent1(part, degp):
    return pl.pallas_call(
        _merge_ent1_body,
        out_shape=[jax.ShapeDtypeStruct((ENT_PAD, EMB), jnp.float32),
                   jax.ShapeDtypeStruct((ENT_PAD, 8), jnp.float32)],
    )(part, degp)


def _merge_ent2_body(part_ref, deg_ref, cur_ref):
    p = part_ref[0] + part_ref[1]
    agg = p / deg_ref[:, :1]
    n = jnp.sqrt(jnp.sum(agg * agg, axis=1, keepdims=True))
    cur_ref[...] = agg / (n + 1e-8)


def _merge_ent2(part, deg):
    return pl.pallas_call(
        _merge_ent2_body,
        out_shape=jax.ShapeDtypeStruct((ENT_PAD, EMB), jnp.float32),
    )(part, deg)


def _merge_user_body(part_ref, cu_ref, ures_ref, latent_ref, att_ref,
                     relw_ref, cu_out_ref, ures_out_ref):
    ua = part_ref[0] + part_ref[1]
    cu_prev = cu_ref[...]
    logits = lax.dot_general(cu_prev, latent_ref[...],
                             (((1,), (1,)), ((), ())),
                             preferred_element_type=jnp.float32)
    m = jnp.max(logits, axis=1, keepdims=True)
    e = jnp.exp(logits - m)
    score = e / jnp.sum(e, axis=1, keepdims=True)
    a = att_ref[...]
    am = jnp.max(a, axis=1, keepdims=True)
    ae = jnp.exp(a - am)
    aw = ae / jnp.sum(ae, axis=1, keepdims=True)
    disen_w = jnp.dot(aw, relw_ref[...], preferred_element_type=jnp.float32)
    scale = jnp.dot(score, disen_w, preferred_element_type=jnp.float32)
    ua = ua * scale + ua
    n = jnp.sqrt(jnp.sum(ua * ua, axis=1, keepdims=True))
    cu = ua / (n + 1e-8)
    cu_out_ref[...] = cu
    ures_out_ref[...] = ures_ref[...] + cu


def _merge_user(part, cu_prev, ures_prev, latent_emb, att, rel_weight):
    return pl.pallas_call(
        _merge_user_body,
        out_shape=[jax.ShapeDtypeStruct((LEAF_PAD, EMB), jnp.float32),
                   jax.ShapeDtypeStruct((LEAF_PAD, EMB), jnp.float32)],
    )(part, cu_prev, ures_prev, latent_emb, att, rel_weight)


def _attn_body(x_ref, lt_ref, l_ref, kmask_ref, tmult_ref, o_ref):
    x = x_ref[...]
    sc = jnp.dot(x, lt_ref[...], preferred_element_type=jnp.float32)
    sc = sc + kmask_ref[...]
    m = jnp.max(sc, axis=1, keepdims=True)
    p = jnp.exp(sc - m)
    den = jnp.sum(p, axis=1, keepdims=True)
    w = p / den
    o = jnp.dot(w, l_ref[...], preferred_element_type=jnp.float32)
    o_ref[...] = o * tmult_ref[...]


def _intent_attention(x, lpad, tok_mult):
    kmask = jnp.where(jnp.arange(LEAF_PAD) < KG_LEAF, 0.0, -1e30)[None, :]
    grid = NTOK // TB
    return pl.pallas_call(
        _attn_body,
        grid=(grid,),
        in_specs=[
            pl.BlockSpec((TB, EMB), lambda i: (i, 0)),
            pl.BlockSpec((EMB, LEAF_PAD), lambda i: (0, 0)),
            pl.BlockSpec((LEAF_PAD, EMB), lambda i: (0, 0)),
            pl.BlockSpec((1, LEAF_PAD), lambda i: (0, 0)),
            pl.BlockSpec((TB, 1), lambda i: (i, 0)),
        ],
        out_specs=pl.BlockSpec((TB, EMB), lambda i: (i, 0)),
        out_shape=jax.ShapeDtypeStruct((NTOK, EMB), jnp.float32),
    )(x, lpad.T, lpad, kmask, tok_mult)


def _pad_chunks(x, nch_pad, fill):
    xx = x if x.dtype == jnp.float32 else x.astype(jnp.int32)
    pad = jnp.full((nch_pad * CH - xx.shape[0],), fill, xx.dtype)
    return jnp.concatenate([xx, pad]).reshape(nch_pad, CH)


def kernel(diag_table, proc_table, med_table, all_embed, latent_emb,
           rel_weight, disen_weight_att, inter_val, d_mask, p_mask, m_mask,
           diseases, procedures, medications, edge_index, edge_type,
           inter_row, inter_col):
    head = edge_index[0]
    tail = edge_index[1]
    tail2 = _pad_chunks(tail, NCHT_E_PAD, N_ENT)
    rel2 = _pad_chunks(edge_type, NCHT_E_PAD, 1)
    head2 = _pad_chunks(head, NCHT_E_PAD, N_ENT)
    col2 = _pad_chunks(inter_col, NCHT_U_PAD, 0)
    row2 = _pad_chunks(inter_row, NCHT_U_PAD, KG_LEAF)
    val_p = jnp.concatenate(
        [inter_val, jnp.zeros((NCHT_U_PAD * CH - NNZ,), jnp.float32)])
    valb = jnp.broadcast_to(val_p[:, None], (NCHT_U_PAD * CH, 16))

    gidx2 = _gidx_kernel(tail2, rel2)

    ent0 = jnp.zeros((ENT_PAD, EMB), jnp.float32).at[:N_ENT].set(
        all_embed[KG_LEAF:])
    user0 = jnp.zeros((LEAF_PAD, EMB), jnp.float32).at[:KG_LEAF].set(
        all_embed[:KG_LEAF])
    zeros = jnp.zeros((ENT_PAD, EMB), jnp.float32)

    edge1 = _make_edge_kernel(True)
    edge2 = _make_edge_kernel(False)

    p0 = _prescale(ent0, rel_weight).reshape((N_REL - 1) * ENT_PAD, EMB)
    e_part1, degp = edge1(p0, gidx2, head2, zeros)
    u_part1 = _user_kernel(ent0, col2, row2, valb, zeros)
    cur_ent1, deg8 = _merge_ent1(e_part1, degp)
    cur_user1, user_res = _merge_user(u_part1, user0, user0, latent_emb,
                                      disen_weight_att, rel_weight)
    p1 = _prescale(cur_ent1, rel_weight).reshape((N_REL - 1) * ENT_PAD, EMB)
    e_part2 = edge2(p1, gidx2, head2, zeros)
    u_part2 = _user_kernel(cur_ent1, col2, row2, valb, zeros)
    cur_ent2 = _merge_ent2(e_part2, deg8)
    cur_user2, user_res = _merge_user(u_part2, cur_user1, user_res,
                                      latent_emb, disen_weight_att,
                                      rel_weight)
    u_part3 = _user_kernel(cur_ent2, col2, row2, valb, zeros)
    _, user_res = _merge_user(u_part3, cur_user2, user_res, latent_emb,
                              disen_weight_att, rel_weight)

    new_med = jnp.concatenate(
        [jnp.full((B, 1, MN), MED_PAD, dtype=medications.dtype),
         medications[:, :-1, :]], axis=1)
    new_m_mask = jnp.concatenate(
        [jnp.full((B, 1, MN), -1e9, dtype=jnp.float32),
         m_mask[:, :-1, :]], axis=1)

    diag_emb = diag_table[diseases].reshape(B * S, DN, EMB)
    proc_emb = proc_table[procedures].reshape(B * S, PN, EMB)
    med_emb = med_table[new_med].reshape(B * S, MN, EMB)
    x = jnp.concatenate([diag_emb, proc_emb, med_emb], axis=1)
    x = x.reshape(NTOK, EMB)

    masks = jnp.concatenate([d_mask.reshape(B * S, DN),
                             p_mask.reshape(B * S, PN),
                             new_m_mask.reshape(B * S, MN)], axis=1)
    tok_mult = jnp.where(masks.reshape(NTOK, 1) != 0, 0.0, 1.0)

    out = _intent_attention(x, user_res, tok_mult)
    return out.reshape(B * S, DN + PN + MN, EMB)
